# Initial kernel scaffold; baseline (speedup 1.0000x reference)
import jax
import jax.numpy as jnp
from jax.experimental import pallas as pl


def kernel(x, edge_index, edge_attr, batch, W1, b1, gamma1, beta1, W2, b2, gamma2, beta2, Wl, bl):
    raise NotImplementedError("write your pallas kernel here")


# speedup vs baseline: 9.9309x; 9.9309x over previous
import functools

import jax
import jax.numpy as jnp
from jax import lax
from jax.experimental import pallas as pl
from jax.experimental.pallas import tpu as pltpu
from jax.experimental.pallas import tpu_sc as plsc

N = 10000
E = 320000
D = 128
G = 64
C = 10
EPS = 1e-5

NP = 10240
ROWS_PT = NP // 16
EDGES_PT = E // 32
CH = 80
NCHUNK = EDGES_PT // CH

_mesh = plsc.VectorSubcoreMesh(core_axis_name="c", subcore_axis_name="s")


@functools.partial(
    pl.kernel,
    mesh=_mesh,
    out_type=jax.ShapeDtypeStruct((2, NP), jnp.float32),
    scratch_types=[
        pltpu.VMEM((CH,), jnp.int32),
        pltpu.VMEM((CH,), jnp.float32),
        pltpu.VMEM_SHARED((NP,), jnp.float32),
    ],
)
def _deg_sc(col_hbm, ew_hbm, zrow_hbm, out_hbm, idx_v, val_v, acc_sh):
    cid = lax.axis_index("c")
    sid = lax.axis_index("s")
    r0 = sid * ROWS_PT
    pltpu.sync_copy(zrow_hbm, acc_sh.at[pl.ds(r0, ROWS_PT)])
    plsc.subcore_barrier()

    base = cid * (E // 2) + sid * EDGES_PT

    def body(i, _):
        off = base + i * CH
        pltpu.sync_copy(col_hbm.at[pl.ds(off, CH)], idx_v)
        pltpu.sync_copy(ew_hbm.at[pl.ds(off, CH)], val_v)
        pltpu.sync_copy(val_v, acc_sh.at[idx_v], add=True)
        return _

    lax.fori_loop(0, NCHUNK, body, None)
    plsc.subcore_barrier()
    pltpu.sync_copy(acc_sh.at[pl.ds(r0, ROWS_PT)], out_hbm.at[cid, pl.ds(r0, ROWS_PT)])


@functools.partial(
    pl.kernel,
    mesh=_mesh,
    out_type=jax.ShapeDtypeStruct((2, NP, D), jnp.float32),
    scratch_types=[
        pltpu.VMEM((CH,), jnp.int32),
        pltpu.VMEM((CH,), jnp.int32),
        pltpu.VMEM((CH,), jnp.float32),
        pltpu.VMEM((CH, D), jnp.float32),
        pltpu.VMEM_SHARED((NP, D), jnp.float32),
        pltpu.SemaphoreType.DMA,
    ],
)
def _msg_sc(table_hbm, row_hbm, col_hbm, ew_hbm, zrows_hbm, out_hbm,
            idxr_v, idxc_v, ew_v, rows_v, acc_sh, sem):
    cid = lax.axis_index("c")
    sid = lax.axis_index("s")
    r0 = sid * ROWS_PT
    pltpu.sync_copy(zrows_hbm, acc_sh.at[pl.ds(r0, ROWS_PT)])
    plsc.subcore_barrier()

    base = cid * (E // 2) + sid * EDGES_PT

    def body(i, _):
        off = base + i * CH
        pltpu.sync_copy(row_hbm.at[pl.ds(off, CH)], idxr_v)
        pltpu.sync_copy(col_hbm.at[pl.ds(off, CH)], idxc_v)
        pltpu.sync_copy(ew_hbm.at[pl.ds(off, CH)], ew_v)
        pltpu.async_copy(table_hbm.at[idxr_v], rows_v, sem).wait()

        def scale(g, _c):
            wv = ew_v[pl.ds(g * 16, 16)]
            for k2 in range(16):
                w = jnp.full((16,), wv[k2], jnp.float32)
                k = g * 16 + k2
                for j in range(D // 16):
                    sl = pl.ds(j * 16, 16)
                    rows_v[k, sl] = rows_v[k, sl] * w
            return _c

        lax.fori_loop(0, CH // 16, scale, None)
        pltpu.sync_copy(rows_v, acc_sh.at[idxc_v], add=True)
        return _

    lax.fori_loop(0, NCHUNK, body, None)
    plsc.subcore_barrier()
    pltpu.sync_copy(acc_sh.at[pl.ds(r0, ROWS_PT)],
                    out_hbm.at[cid, pl.ds(r0, ROWS_PT)])


def _tc1_body(x_ref, w1_ref, degp_ref, h1s_ref, dinv_ref):
    deg = degp_ref[0, :N] + degp_ref[1, :N] + 1.0
    dinv = jnp.where(deg > 0, lax.rsqrt(deg), 0.0)
    h1 = jnp.dot(x_ref[...], w1_ref[...], preferred_element_type=jnp.float32)
    h1s_ref[...] = h1 * dinv[:, None]
    dinv_ref[...] = dinv[:, None]


def _tc2_body(sp_ref, hs_ref, dinv_ref, b_ref, g_ref, be_ref, w2_ref, out_ref):
    s = sp_ref[0, :N, :] + sp_ref[1, :N, :]
    dinv = dinv_ref[...]
    z = dinv * (s + hs_ref[...]) + b_ref[...]
    mu = jnp.mean(z, axis=0, keepdims=True)
    var = jnp.mean((z - mu) * (z - mu), axis=0, keepdims=True)
    zn = (z - mu) * lax.rsqrt(var + EPS) * g_ref[...] + be_ref[...]
    h = jnp.maximum(zn, 0.0)
    h2 = jnp.dot(h, w2_ref[...], preferred_element_type=jnp.float32)
    out_ref[...] = h2 * dinv


def _tc3_body(sp_ref, hs_ref, dinv_ref, b_ref, g_ref, be_ref, batch_ref,
              wl_ref, bl_ref, out_ref):
    s = sp_ref[0, :N, :] + sp_ref[1, :N, :]
    dinv = dinv_ref[...]
    z = dinv * (s + hs_ref[...]) + b_ref[...]
    mu = jnp.mean(z, axis=0, keepdims=True)
    var = jnp.mean((z - mu) * (z - mu), axis=0, keepdims=True)
    zn = (z - mu) * lax.rsqrt(var + EPS) * g_ref[...] + be_ref[...]
    h = jnp.maximum(zn, 0.0)
    gi = lax.broadcasted_iota(jnp.int32, (N, G), 1)
    oh = (batch_ref[...] == gi).astype(jnp.float32)
    cnt = jnp.sum(oh, axis=0)
    ssum = lax.dot_general(oh, h, (((0,), (0,)), ((), ())),
                           preferred_element_type=jnp.float32)
    pooled = ssum / jnp.maximum(cnt, 1.0)[:, None]
    out_ref[...] = jnp.dot(pooled, wl_ref[...],
                           preferred_element_type=jnp.float32) + bl_ref[...]


def kernel(x, edge_index, edge_attr, batch, W1, b1, gamma1, beta1,
           W2, b2, gamma2, beta2, Wl, bl):
    row = edge_index[0]
    col = edge_index[1]
    zrow = jnp.zeros((ROWS_PT,), jnp.float32)
    zrows = jnp.zeros((ROWS_PT, D), jnp.float32)

    degp = _deg_sc(col, edge_attr, zrow)

    h1s, dinv = pl.pallas_call(
        _tc1_body,
        out_shape=[jax.ShapeDtypeStruct((N, D), jnp.float32),
                   jax.ShapeDtypeStruct((N, 1), jnp.float32)],
    )(x, W1, degp)

    s1p = _msg_sc(h1s, row, col, edge_attr, zrows)

    h2s = pl.pallas_call(
        _tc2_body,
        out_shape=jax.ShapeDtypeStruct((N, D), jnp.float32),
    )(s1p, h1s, dinv, b1[None, :], gamma1[None, :], beta1[None, :], W2)

    s2p = _msg_sc(h2s, row, col, edge_attr, zrows)

    out = pl.pallas_call(
        _tc3_body,
        out_shape=jax.ShapeDtypeStruct((G, C), jnp.float32),
    )(s2p, h2s, dinv, b2[None, :], gamma2[None, :], beta2[None, :],
      batch[:, None], Wl, bl[None, :])
    return out
